# Initial kernel scaffold; baseline (speedup 1.0000x reference)
import jax
import jax.numpy as jnp
from jax.experimental import pallas as pl


def kernel(x, adj, W1, b1, W2, b2, W3, b3, Wfc, bfc):
    raise NotImplementedError("write your pallas kernel here")


# speedup vs baseline: 1.6895x; 1.6895x over previous
import jax
import jax.numpy as jnp
from jax.experimental import pallas as pl


def _mm_kernel(a_ref, b_ref, o_ref):
    o_ref[...] = jnp.dot(a_ref[...], b_ref[...],
                         preferred_element_type=jnp.float32)


def _propagate(adj, rhs, bm):
    m, n = adj.shape
    k = rhs.shape[1]
    return pl.pallas_call(
        _mm_kernel,
        grid=(m // bm,),
        in_specs=[
            pl.BlockSpec((bm, n), lambda i: (i, 0)),
            pl.BlockSpec((n, k), lambda i: (0, 0)),
        ],
        out_specs=pl.BlockSpec((bm, k), lambda i: (i, 0)),
        out_shape=jax.ShapeDtypeStruct((m, k), jnp.float32),
    )(adj, rhs)


def _final_kernel(adj_ref, v3_ref, u1_ref, v2_ref, bcat_ref, wfc_ref, bfc_ref,
                  o_ref):
    w3 = jnp.dot(adj_ref[...], v3_ref[...],
                 preferred_element_type=jnp.float32)
    h = jnp.concatenate([u1_ref[...], v2_ref[...], w3], axis=1)
    h = jax.nn.relu(h + bcat_ref[...])
    logits = jnp.dot(h, wfc_ref[...], preferred_element_type=jnp.float32)
    o_ref[...] = jax.nn.sigmoid(logits + bfc_ref[...])


def _final_pass(adj, v3, u1, v2, bcat, wfc, bfc, bm):
    m, n = adj.shape
    k = v3.shape[1]
    kh = bcat.shape[1]
    nl = wfc.shape[1]
    return pl.pallas_call(
        _final_kernel,
        grid=(m // bm,),
        in_specs=[
            pl.BlockSpec((bm, n), lambda i: (i, 0)),
            pl.BlockSpec((n, k), lambda i: (0, 0)),
            pl.BlockSpec((bm, k), lambda i: (i, 0)),
            pl.BlockSpec((bm, k), lambda i: (i, 0)),
            pl.BlockSpec((1, kh), lambda i: (0, 0)),
            pl.BlockSpec((kh, nl), lambda i: (0, 0)),
            pl.BlockSpec((1, nl), lambda i: (0, 0)),
        ],
        out_specs=pl.BlockSpec((bm, nl), lambda i: (i, 0)),
        out_shape=jax.ShapeDtypeStruct((m, nl), jnp.float32),
    )(adj, v3, u1, v2, bcat, wfc, bfc)


def _pick_bm(m):
    for bm in (400, 200, 80, 40, 16, 8):
        if m % bm == 0:
            return bm
    return m


def kernel(x, adj, W1, b1, W2, b2, W3, b3, Wfc, bfc):
    m = adj.shape[0]
    nh = W1.shape[1]
    bm = _pick_bm(m)

    wcat = jnp.concatenate([W1, W2, W3], axis=1)
    bcat = jnp.concatenate([b1, b2, b3])[None, :]

    t = _propagate(x, wcat, bm)
    u = _propagate(adj, t, bm)
    v = _propagate(adj, u[:, nh:], bm)
    out = _final_pass(adj, v[:, nh:], u[:, :nh], v[:, :nh],
                      bcat, Wfc, bfc[None, :], bm)
    return out


# speedup vs baseline: 1.7677x; 1.0463x over previous
import jax
import jax.numpy as jnp
from jax.experimental import pallas as pl


def _mm_kernel(a_ref, b_ref, o_ref):
    o_ref[...] = jnp.dot(a_ref[...], b_ref[...],
                         preferred_element_type=jnp.float32)


def _mm_split_kernel(a_ref, b_ref, o1_ref, o2_ref):
    prod = jnp.dot(a_ref[...], b_ref[...], preferred_element_type=jnp.float32)
    k1 = o1_ref.shape[1]
    o1_ref[...] = prod[:, :k1]
    o2_ref[...] = prod[:, k1:]


def _propagate(adj, rhs, bm):
    m, n = adj.shape
    k = rhs.shape[1]
    return pl.pallas_call(
        _mm_kernel,
        grid=(m // bm,),
        in_specs=[
            pl.BlockSpec((bm, n), lambda i: (i, 0)),
            pl.BlockSpec((n, k), lambda i: (0, 0)),
        ],
        out_specs=pl.BlockSpec((bm, k), lambda i: (i, 0)),
        out_shape=jax.ShapeDtypeStruct((m, k), jnp.float32),
    )(adj, rhs)


def _propagate_split(adj, rhs, k1, bm):
    m, n = adj.shape
    k = rhs.shape[1]
    return pl.pallas_call(
        _mm_split_kernel,
        grid=(m // bm,),
        in_specs=[
            pl.BlockSpec((bm, n), lambda i: (i, 0)),
            pl.BlockSpec((n, k), lambda i: (0, 0)),
        ],
        out_specs=[
            pl.BlockSpec((bm, k1), lambda i: (i, 0)),
            pl.BlockSpec((bm, k - k1), lambda i: (i, 0)),
        ],
        out_shape=[
            jax.ShapeDtypeStruct((m, k1), jnp.float32),
            jax.ShapeDtypeStruct((m, k - k1), jnp.float32),
        ],
    )(adj, rhs)


def _final_kernel(adj_ref, v3_ref, u1_ref, v2_ref, bcat_ref, wfc_ref, bfc_ref,
                  o_ref):
    w3 = jnp.dot(adj_ref[...], v3_ref[...],
                 preferred_element_type=jnp.float32)
    h = jnp.concatenate([u1_ref[...], v2_ref[...], w3], axis=1)
    h = jax.nn.relu(h + bcat_ref[...])
    logits = jnp.dot(h, wfc_ref[...], preferred_element_type=jnp.float32)
    o_ref[...] = jax.nn.sigmoid(logits + bfc_ref[...])


def _final_pass(adj, v3, u1, v2, bcat, wfc, bfc, bm):
    m, n = adj.shape
    k = v3.shape[1]
    kh = bcat.shape[1]
    nl = wfc.shape[1]
    return pl.pallas_call(
        _final_kernel,
        grid=(m // bm,),
        in_specs=[
            pl.BlockSpec((bm, n), lambda i: (i, 0)),
            pl.BlockSpec((n, k), lambda i: (0, 0)),
            pl.BlockSpec((bm, k), lambda i: (i, 0)),
            pl.BlockSpec((bm, k), lambda i: (i, 0)),
            pl.BlockSpec((1, kh), lambda i: (0, 0)),
            pl.BlockSpec((kh, nl), lambda i: (0, 0)),
            pl.BlockSpec((1, nl), lambda i: (0, 0)),
        ],
        out_specs=pl.BlockSpec((bm, nl), lambda i: (i, 0)),
        out_shape=jax.ShapeDtypeStruct((m, nl), jnp.float32),
    )(adj, v3, u1, v2, bcat, wfc, bfc)


def _pick_bm(m):
    for bm in (400, 200, 80, 40, 16, 8):
        if m % bm == 0:
            return bm
    return m


def kernel(x, adj, W1, b1, W2, b2, W3, b3, Wfc, bfc):
    m = adj.shape[0]
    nh = W1.shape[1]
    bm = _pick_bm(m)

    wcat = jnp.concatenate([W1, W2, W3], axis=1)
    bcat = jnp.concatenate([b1, b2, b3])[None, :]

    t = _propagate(x, wcat, bm)
    u1, u23 = _propagate_split(adj, t, nh, bm)
    v2, v3 = _propagate_split(adj, u23, nh, bm)
    out = _final_pass(adj, v3, u1, v2,
                      bcat, Wfc, bfc[None, :], bm)
    return out


# speedup vs baseline: 1.9596x; 1.1085x over previous
import jax
import jax.numpy as jnp
from jax.experimental import pallas as pl


def _mm_kernel(a_ref, b_ref, o_ref):
    o_ref[...] = jnp.dot(a_ref[...], b_ref[...],
                         preferred_element_type=jnp.float32)


def _propagate(adj, rhs, bm):
    m, n = adj.shape
    k = rhs.shape[1]
    return pl.pallas_call(
        _mm_kernel,
        grid=(m // bm,),
        in_specs=[
            pl.BlockSpec((bm, n), lambda i: (i, 0)),
            pl.BlockSpec((n, k), lambda i: (0, 0)),
        ],
        out_specs=pl.BlockSpec((bm, k), lambda i: (i, 0)),
        out_shape=jax.ShapeDtypeStruct((m, k), jnp.float32),
    )(adj, rhs)


def _final_kernel(adj_ref, y1_ref, y2_ref, wcat_ref, w3_ref, bcat_ref,
                  wfc_ref, bfc_ref, o_ref):
    bm = y1_ref.shape[0]
    i = pl.program_id(0)
    y3 = jnp.dot(adj_ref[...], y2_ref[...],
                 preferred_element_type=jnp.float32)
    y2_blk = y2_ref[pl.ds(i * bm, bm), :]
    y12 = jnp.concatenate([y1_ref[...], y2_blk], axis=1)
    h12 = jnp.dot(y12, wcat_ref[...], preferred_element_type=jnp.float32)
    h3 = jnp.dot(y3, w3_ref[...], preferred_element_type=jnp.float32)
    h = jax.nn.relu(jnp.concatenate([h12, h3], axis=1) + bcat_ref[...])
    logits = jnp.dot(h, wfc_ref[...], preferred_element_type=jnp.float32)
    o_ref[...] = jax.nn.sigmoid(logits + bfc_ref[...])


def _final_pass(adj, y1, y2, wcat, w3, bcat, wfc, bfc, bm):
    m, n = adj.shape
    k = y1.shape[1]
    kh = bcat.shape[1]
    nl = wfc.shape[1]
    return pl.pallas_call(
        _final_kernel,
        grid=(m // bm,),
        in_specs=[
            pl.BlockSpec((bm, n), lambda i: (i, 0)),
            pl.BlockSpec((bm, k), lambda i: (i, 0)),
            pl.BlockSpec((n, k), lambda i: (0, 0)),
            pl.BlockSpec((2 * k, kh - k), lambda i: (0, 0)),
            pl.BlockSpec((k, k), lambda i: (0, 0)),
            pl.BlockSpec((1, kh), lambda i: (0, 0)),
            pl.BlockSpec((kh, nl), lambda i: (0, 0)),
            pl.BlockSpec((1, nl), lambda i: (0, 0)),
        ],
        out_specs=pl.BlockSpec((bm, nl), lambda i: (i, 0)),
        out_shape=jax.ShapeDtypeStruct((m, nl), jnp.float32),
    )(adj, y1, y2, wcat, w3, bcat, wfc, bfc)


def _pick_bm(m):
    for bm in (400, 200, 80, 40, 16, 8):
        if m % bm == 0:
            return bm
    return m


def kernel(x, adj, W1, b1, W2, b2, W3, b3, Wfc, bfc):
    m = adj.shape[0]
    nh = W1.shape[1]
    bm = _pick_bm(m)

    zeros = jnp.zeros_like(W1)
    wcat = jnp.block([[W1, zeros], [zeros, W2]])
    bcat = jnp.concatenate([b1, b2, b3])[None, :]

    y1 = _propagate(adj, x, bm)
    y2 = _propagate(adj, y1, bm)
    out = _final_pass(adj, y1, y2, wcat, W3, bcat, Wfc, bfc[None, :], bm)
    return out


# speedup vs baseline: 2.0297x; 1.0358x over previous
import jax
import jax.numpy as jnp
from jax.experimental import pallas as pl
from jax.experimental.pallas import tpu as pltpu


def _ngcn_kernel(adj_ref, x_ref, wcat_ref, w3_ref, bcat_ref, wfc_ref,
                 bfc_ref, o_ref, y1_scr, y2_scr):
    p = pl.program_id(0)
    i = pl.program_id(1)
    bm = adj_ref.shape[0]

    @pl.when(p == 0)
    def _hop1():
        y1_scr[pl.ds(i * bm, bm), :] = jnp.dot(
            adj_ref[...], x_ref[...], preferred_element_type=jnp.float32)

    @pl.when(p == 1)
    def _hop2():
        y2_scr[pl.ds(i * bm, bm), :] = jnp.dot(
            adj_ref[...], y1_scr[...], preferred_element_type=jnp.float32)

    @pl.when(p == 2)
    def _hop3_epilogue():
        y3 = jnp.dot(adj_ref[...], y2_scr[...],
                     preferred_element_type=jnp.float32)
        y12 = jnp.concatenate(
            [y1_scr[pl.ds(i * bm, bm), :], y2_scr[pl.ds(i * bm, bm), :]],
            axis=1)
        h12 = jnp.dot(y12, wcat_ref[...], preferred_element_type=jnp.float32)
        h3 = jnp.dot(y3, w3_ref[...], preferred_element_type=jnp.float32)
        h = jax.nn.relu(jnp.concatenate([h12, h3], axis=1) + bcat_ref[...])
        logits = jnp.dot(h, wfc_ref[...], preferred_element_type=jnp.float32)
        o_ref[...] = jax.nn.sigmoid(logits + bfc_ref[...])


def _pick_bm(m):
    for bm in (400, 200, 80, 40, 16, 8):
        if m % bm == 0:
            return bm
    return m


def kernel(x, adj, W1, b1, W2, b2, W3, b3, Wfc, bfc):
    m, n = adj.shape
    nh = W1.shape[1]
    nl = Wfc.shape[1]
    kh = Wfc.shape[0]
    bm = _pick_bm(m)

    zeros = jnp.zeros_like(W1)
    wcat = jnp.block([[W1, zeros], [zeros, W2]])
    bcat = jnp.concatenate([b1, b2, b3])[None, :]

    return pl.pallas_call(
        _ngcn_kernel,
        grid=(3, m // bm),
        in_specs=[
            pl.BlockSpec((bm, n), lambda p, i: (i, 0)),
            pl.BlockSpec((n, nh), lambda p, i: (0, 0)),
            pl.BlockSpec((2 * nh, 2 * nh), lambda p, i: (0, 0)),
            pl.BlockSpec((nh, nh), lambda p, i: (0, 0)),
            pl.BlockSpec((1, kh), lambda p, i: (0, 0)),
            pl.BlockSpec((kh, nl), lambda p, i: (0, 0)),
            pl.BlockSpec((1, nl), lambda p, i: (0, 0)),
        ],
        out_specs=pl.BlockSpec((bm, nl), lambda p, i: (i, 0)),
        out_shape=jax.ShapeDtypeStruct((m, nl), jnp.float32),
        scratch_shapes=[
            pltpu.VMEM((m, nh), jnp.float32),
            pltpu.VMEM((m, nh), jnp.float32),
        ],
    )(adj, x, wcat, W3, bcat, Wfc, bfc[None, :])
